# baseline (speedup 1.0000x reference)
import functools

import jax
import jax.numpy as jnp
from jax import lax
from jax.experimental import pallas as pl
from jax.experimental.pallas import tpu as pltpu
from jax.experimental.pallas import tpu_sc as plsc

B = 16384
F = 100
VOCAB = 1000000
VOCAB_PAD = 1000448

NC = 2
NS = 16
NW = NC * NS

W = 128
EPC = W * F
NCHUNK = B // W
CPW = NCHUNK // NW

TAB_SLICE = VOCAB_PAD // NS


def _body(idx_hbm, val_hbm, bias_hbm, table_hbm, out_hbm,
          idx_v0, idx_v1, val_v0, val_v1, gat_v, out_v, bias_v, tab_s,
          si0, si1, sv0, sv1, sg, sa, sb):
  sid = lax.axis_index("s")
  wid = sid * NC + lax.axis_index("c")
  pltpu.sync_copy(bias_hbm, bias_v)
  bias_vec = bias_v[...]
  idx_bufs = (idx_v0, idx_v1)
  val_bufs = (val_v0, val_v1)
  idx_sems = (si0, si1)
  val_sems = (sv0, sv1)

  def idx_start(k, s):
    c0 = (wid * CPW + k) * W
    ib = idx_bufs[s]

    def issue(f, carry):
      pltpu.make_async_copy(
          idx_hbm.at[f, pl.ds(c0, W)], ib.at[pl.ds(f * W, W)], idx_sems[s]
      ).start()
      return carry

    lax.fori_loop(0, F, issue, 0)

  def val_start(k, s):
    c0 = (wid * CPW + k) * W
    vb = val_bufs[s]

    def issue(f, carry):
      pltpu.make_async_copy(
          val_hbm.at[f, pl.ds(c0, W)], vb.at[pl.ds(f * W, W)], val_sems[s]
      ).start()
      return carry

    lax.fori_loop(0, F, issue, 0)

  def copies_start(k, s):
    idx_start(k, s)
    val_start(k, s)

  def copies_wait(k, s):
    c0 = (wid * CPW + k) * W
    ib, vb = idx_bufs[s], val_bufs[s]

    def drain(f, carry):
      pltpu.make_async_copy(
          idx_hbm.at[f, pl.ds(c0, W)], ib.at[pl.ds(f * W, W)], idx_sems[s]
      ).wait()
      pltpu.make_async_copy(
          val_hbm.at[f, pl.ds(c0, W)], vb.at[pl.ds(f * W, W)], val_sems[s]
      ).wait()
      return carry

    lax.fori_loop(0, F, drain, 0)

  idx_start(0, 0)

  base = sid * TAB_SLICE
  pieces = [EPC] * 4 + [TAB_SLICE - 4 * EPC]
  offs = [sum(pieces[:i]) for i in range(len(pieces))]
  bounce = (gat_v, val_bufs[0], val_bufs[1])

  def _arr(i):
    return pltpu.make_async_copy(
        table_hbm.at[pl.ds(base + offs[i], pieces[i])],
        bounce[i % 3].at[pl.ds(0, pieces[i])], sa)

  def _wr(i):
    return pltpu.make_async_copy(
        bounce[i % 3].at[pl.ds(0, pieces[i])],
        tab_s.at[pl.ds(base + offs[i], pieces[i])], sb)

  n = len(pieces)
  for i in range(3):
    _arr(i).start()
  for i in range(n):
    _arr(i).wait()
    _wr(i).start()
    if i + 3 < n:
      _wr(i).wait()
      _arr(i + 3).start()
  for i in range(max(0, n - 3), n):
    _wr(i).wait()
  plsc.subcore_barrier()

  val_start(0, 0)
  for k in range(CPW):
    s = k % 2
    copies_wait(k, s)
    gather = pltpu.make_async_copy(tab_s.at[idx_bufs[s]], gat_v, sg)
    gather.start()
    if k + 1 < CPW:
      copies_start(k + 1, 1 - s)
    gather.wait()

    vb = val_bufs[s]

    def fbody(f, accs):
      out = []
      for g in range(8):
        a = gat_v[pl.ds(f * W + g * 16, 16)]
        v = vb[pl.ds(f * W + g * 16, 16)]
        out.append(accs[g] + a * v)
      return tuple(out)

    accs = lax.fori_loop(
        0, F, fbody, tuple(jnp.zeros((16,), jnp.float32) for _ in range(8))
    )
    for g in range(8):
      out_v[pl.ds(g * 16, 16)] = accs[g] + bias_vec

    c0 = (wid * CPW + k) * W
    pltpu.sync_copy(out_v, out_hbm.at[pl.ds(c0, W)])


@jax.jit
def _wide_sc(idx, val, bias16, tab):
  mesh = plsc.VectorSubcoreMesh(core_axis_name="c", subcore_axis_name="s")
  f = pl.kernel(
      _body,
      mesh=mesh,
      compiler_params=pltpu.CompilerParams(needs_layout_passes=False),
      out_type=jax.ShapeDtypeStruct((B,), jnp.float32),
      scratch_types=[
          pltpu.VMEM((EPC,), jnp.int32),
          pltpu.VMEM((EPC,), jnp.int32),
          pltpu.VMEM((EPC,), jnp.float32),
          pltpu.VMEM((EPC,), jnp.float32),
          pltpu.VMEM((EPC,), jnp.float32),
          pltpu.VMEM((W,), jnp.float32),
          pltpu.VMEM((16,), jnp.float32),
          pltpu.VMEM_SHARED((VOCAB_PAD,), jnp.float32),
          pltpu.SemaphoreType.DMA,
          pltpu.SemaphoreType.DMA,
          pltpu.SemaphoreType.DMA,
          pltpu.SemaphoreType.DMA,
          pltpu.SemaphoreType.DMA,
          pltpu.SemaphoreType.DMA,
          pltpu.SemaphoreType.DMA,
      ],
  )
  return f(idx, val, bias16, tab)


def kernel(index, field, value, table, bias):
  del field
  idx = index.astype(jnp.int32).T
  val = value.T
  tab = jnp.pad(table, ((0, VOCAB_PAD - VOCAB), (0, 0)))[:, 0]
  bias16 = jnp.broadcast_to(bias.astype(jnp.float32), (16,))
  out = _wide_sc(idx, val, bias16, tab)
  return out.reshape(B, 1)


# speedup vs baseline: 1.0326x; 1.0326x over previous
import functools

import jax
import jax.numpy as jnp
from jax import lax
from jax.experimental import pallas as pl
from jax.experimental.pallas import tpu as pltpu
from jax.experimental.pallas import tpu_sc as plsc

B = 16384
F = 100
VOCAB = 1000000
VOCAB_PAD = 1000448

NC = 2
NS = 16
NW = NC * NS

W = 128
EPC = W * F
NCHUNK = B // W
CPW = NCHUNK // NW

TAB_SLICE = VOCAB_PAD // NS


def _body(idx_hbm, val_hbm, bias_hbm, table_hbm, out_hbm,
          idx_v0, idx_v1, val_v0, val_v1, gat_v, out_v, bias_v, tab_s,
          si0, si1, sv0, sv1, sg, sa, sb):
  sid = lax.axis_index("s")
  wid = sid * NC + lax.axis_index("c")
  pltpu.sync_copy(bias_hbm, bias_v)
  bias_vec = bias_v[...]
  idx_bufs = (idx_v0, idx_v1)
  val_bufs = (val_v0, val_v1)
  idx_sems = (si0, si1)
  val_sems = (sv0, sv1)

  def idx_start(k, s):
    c0 = (wid * CPW + k) * W
    ib = idx_bufs[s]

    def issue(f, carry):
      pltpu.make_async_copy(
          idx_hbm.at[f, pl.ds(c0, W)], ib.at[pl.ds(f * W, W)], idx_sems[s]
      ).start()
      return carry

    lax.fori_loop(0, F, issue, 0)

  def val_start(k, s):
    c0 = (wid * CPW + k) * W
    vb = val_bufs[s]

    def issue(f, carry):
      pltpu.make_async_copy(
          val_hbm.at[f, pl.ds(c0, W)], vb.at[pl.ds(f * W, W)], val_sems[s]
      ).start()
      return carry

    lax.fori_loop(0, F, issue, 0)

  def copies_start(k, s):
    idx_start(k, s)
    val_start(k, s)

  def idx_wait(k, s):
    c0 = (wid * CPW + k) * W
    ib = idx_bufs[s]

    def drain(f, carry):
      pltpu.make_async_copy(
          idx_hbm.at[f, pl.ds(c0, W)], ib.at[pl.ds(f * W, W)], idx_sems[s]
      ).wait()
      return carry

    lax.fori_loop(0, F, drain, 0)

  def val_wait(k, s):
    c0 = (wid * CPW + k) * W
    vb = val_bufs[s]

    def drain(f, carry):
      pltpu.make_async_copy(
          val_hbm.at[f, pl.ds(c0, W)], vb.at[pl.ds(f * W, W)], val_sems[s]
      ).wait()
      return carry

    lax.fori_loop(0, F, drain, 0)

  idx_start(0, 0)

  base = sid * TAB_SLICE
  pieces = [EPC] * 4 + [TAB_SLICE - 4 * EPC]
  offs = [sum(pieces[:i]) for i in range(len(pieces))]
  bounce = (gat_v, val_bufs[0], val_bufs[1])

  def _arr(i):
    return pltpu.make_async_copy(
        table_hbm.at[pl.ds(base + offs[i], pieces[i])],
        bounce[i % 3].at[pl.ds(0, pieces[i])], sa)

  def _wr(i):
    return pltpu.make_async_copy(
        bounce[i % 3].at[pl.ds(0, pieces[i])],
        tab_s.at[pl.ds(base + offs[i], pieces[i])], sb)

  n = len(pieces)
  for i in range(3):
    _arr(i).start()
  for i in range(n):
    _arr(i).wait()
    _wr(i).start()
    if i + 3 < n:
      _wr(i).wait()
      _arr(i + 3).start()
  for i in range(max(0, n - 3), n):
    _wr(i).wait()
  plsc.subcore_barrier()

  val_start(0, 0)
  HALF = EPC // 2
  for k in range(CPW):
    s = k % 2
    ib, vb = idx_bufs[s], val_bufs[s]
    idx_wait(k, s)
    ga = pltpu.make_async_copy(
        tab_s.at[ib.at[pl.ds(0, HALF)]], gat_v.at[pl.ds(0, HALF)], sg)
    ga.start()
    if k + 1 < CPW:
      copies_start(k + 1, 1 - s)
    val_wait(k, s)
    ga.wait()
    gb = pltpu.make_async_copy(
        tab_s.at[ib.at[pl.ds(HALF, HALF)]], gat_v.at[pl.ds(HALF, HALF)], sg)
    gb.start()

    def fbody(f, accs):
      out = []
      for g in range(8):
        a = gat_v[pl.ds(f * W + g * 16, 16)]
        v = vb[pl.ds(f * W + g * 16, 16)]
        out.append(accs[g] + a * v)
      return tuple(out)

    accs = lax.fori_loop(
        0, F // 2, fbody, tuple(jnp.zeros((16,), jnp.float32) for _ in range(8))
    )
    gb.wait()
    accs = lax.fori_loop(F // 2, F, fbody, accs)
    for g in range(8):
      out_v[pl.ds(g * 16, 16)] = accs[g] + bias_vec

    c0 = (wid * CPW + k) * W
    pltpu.sync_copy(out_v, out_hbm.at[pl.ds(c0, W)])


@jax.jit
def _wide_sc(idx, val, bias16, tab):
  mesh = plsc.VectorSubcoreMesh(core_axis_name="c", subcore_axis_name="s")
  f = pl.kernel(
      _body,
      mesh=mesh,
      compiler_params=pltpu.CompilerParams(needs_layout_passes=False),
      out_type=jax.ShapeDtypeStruct((B,), jnp.float32),
      scratch_types=[
          pltpu.VMEM((EPC,), jnp.int32),
          pltpu.VMEM((EPC,), jnp.int32),
          pltpu.VMEM((EPC,), jnp.float32),
          pltpu.VMEM((EPC,), jnp.float32),
          pltpu.VMEM((EPC,), jnp.float32),
          pltpu.VMEM((W,), jnp.float32),
          pltpu.VMEM((16,), jnp.float32),
          pltpu.VMEM_SHARED((VOCAB_PAD,), jnp.float32),
          pltpu.SemaphoreType.DMA,
          pltpu.SemaphoreType.DMA,
          pltpu.SemaphoreType.DMA,
          pltpu.SemaphoreType.DMA,
          pltpu.SemaphoreType.DMA,
          pltpu.SemaphoreType.DMA,
          pltpu.SemaphoreType.DMA,
      ],
  )
  return f(idx, val, bias16, tab)


def kernel(index, field, value, table, bias):
  del field
  idx = index.astype(jnp.int32).T
  val = value.T
  tab = jnp.pad(table, ((0, VOCAB_PAD - VOCAB), (0, 0)))[:, 0]
  bias16 = jnp.broadcast_to(bias.astype(jnp.float32), (16,))
  out = _wide_sc(idx, val, bias16, tab)
  return out.reshape(B, 1)


# speedup vs baseline: 1.0728x; 1.0389x over previous
import functools

import jax
import jax.numpy as jnp
from jax import lax
from jax.experimental import pallas as pl
from jax.experimental.pallas import tpu as pltpu
from jax.experimental.pallas import tpu_sc as plsc

B = 16384
F = 100
VOCAB = 1000000
VOCAB_PAD = 1000448

NC = 2
NS = 16
NW = NC * NS

W = 128
EPC = W * F
NCHUNK = B // W
CPW = NCHUNK // NW

TAB_SLICE = VOCAB_PAD // NS


def _body(idx_hbm, val_hbm, bias_hbm, table_hbm, out_hbm,
          idx_v0, idx_v1, val_v0, val_v1, gat_v, out_v, bias_v, tab_s,
          si0, si1, sv0, sv1, sg, sa, sb):
  sid = lax.axis_index("s")
  wid = sid * NC + lax.axis_index("c")
  pltpu.sync_copy(bias_hbm, bias_v)
  bias_vec = bias_v[...]
  idx_bufs = (idx_v0, idx_v1)
  val_bufs = (val_v0, val_v1)
  idx_sems = (si0, si1)
  val_sems = (sv0, sv1)

  def idx_start(k, s):
    c0 = (wid * CPW + k) * W
    ib = idx_bufs[s]

    def issue(f, carry):
      pltpu.make_async_copy(
          idx_hbm.at[f, pl.ds(c0, W)], ib.at[pl.ds(f * W, W)], idx_sems[s]
      ).start()
      return carry

    lax.fori_loop(0, F, issue, 0)

  def val_start(k, s):
    c0 = (wid * CPW + k) * W
    vb = val_bufs[s]

    def issue(f, carry):
      pltpu.make_async_copy(
          val_hbm.at[f, pl.ds(c0, W)], vb.at[pl.ds(f * W, W)], val_sems[s]
      ).start()
      return carry

    lax.fori_loop(0, F, issue, 0)

  def copies_start(k, s):
    idx_start(k, s)
    val_start(k, s)

  def idx_wait(k, s):
    pltpu.make_async_copy(
        idx_hbm.at[0, pl.ds(0, EPC)], idx_bufs[s], idx_sems[s]
    ).wait()

  def val_wait(k, s):
    pltpu.make_async_copy(
        val_hbm.at[0, pl.ds(0, EPC)], val_bufs[s], val_sems[s]
    ).wait()

  idx_start(0, 0)

  base = sid * TAB_SLICE
  pieces = [EPC] * 4 + [TAB_SLICE - 4 * EPC]
  offs = [sum(pieces[:i]) for i in range(len(pieces))]
  bounce = (gat_v, val_bufs[0], val_bufs[1])

  def _arr(i):
    return pltpu.make_async_copy(
        table_hbm.at[pl.ds(base + offs[i], pieces[i])],
        bounce[i % 3].at[pl.ds(0, pieces[i])], sa)

  def _wr(i):
    return pltpu.make_async_copy(
        bounce[i % 3].at[pl.ds(0, pieces[i])],
        tab_s.at[pl.ds(base + offs[i], pieces[i])], sb)

  n = len(pieces)
  for i in range(3):
    _arr(i).start()
  for i in range(n):
    _arr(i).wait()
    _wr(i).start()
    if i + 3 < n:
      _wr(i).wait()
      _arr(i + 3).start()
  for i in range(max(0, n - 3), n):
    _wr(i).wait()
  plsc.subcore_barrier()

  val_start(0, 0)
  HALF = EPC // 2
  for k in range(CPW):
    s = k % 2
    ib, vb = idx_bufs[s], val_bufs[s]
    idx_wait(k, s)
    ga = pltpu.make_async_copy(
        tab_s.at[ib.at[pl.ds(0, HALF)]], gat_v.at[pl.ds(0, HALF)], sg)
    ga.start()
    if k + 1 < CPW:
      copies_start(k + 1, 1 - s)
    val_wait(k, s)
    ga.wait()
    gb = pltpu.make_async_copy(
        tab_s.at[ib.at[pl.ds(HALF, HALF)]], gat_v.at[pl.ds(HALF, HALF)], sg)
    gb.start()

    def fbody(f, accs):
      out = []
      for g in range(8):
        a = gat_v[pl.ds(f * W + g * 16, 16)]
        v = vb[pl.ds(f * W + g * 16, 16)]
        out.append(accs[g] + a * v)
      return tuple(out)

    accs = lax.fori_loop(
        0, F // 2, fbody, tuple(jnp.zeros((16,), jnp.float32) for _ in range(8))
    )
    gb.wait()
    accs = lax.fori_loop(F // 2, F, fbody, accs)
    for g in range(8):
      out_v[pl.ds(g * 16, 16)] = accs[g] + bias_vec

    c0 = (wid * CPW + k) * W
    pltpu.sync_copy(out_v, out_hbm.at[pl.ds(c0, W)])


@jax.jit
def _wide_sc(idx, val, bias16, tab):
  mesh = plsc.VectorSubcoreMesh(core_axis_name="c", subcore_axis_name="s")
  f = pl.kernel(
      _body,
      mesh=mesh,
      compiler_params=pltpu.CompilerParams(needs_layout_passes=False),
      out_type=jax.ShapeDtypeStruct((B,), jnp.float32),
      scratch_types=[
          pltpu.VMEM((EPC,), jnp.int32),
          pltpu.VMEM((EPC,), jnp.int32),
          pltpu.VMEM((EPC,), jnp.float32),
          pltpu.VMEM((EPC,), jnp.float32),
          pltpu.VMEM((EPC,), jnp.float32),
          pltpu.VMEM((W,), jnp.float32),
          pltpu.VMEM((16,), jnp.float32),
          pltpu.VMEM_SHARED((VOCAB_PAD,), jnp.float32),
          pltpu.SemaphoreType.DMA,
          pltpu.SemaphoreType.DMA,
          pltpu.SemaphoreType.DMA,
          pltpu.SemaphoreType.DMA,
          pltpu.SemaphoreType.DMA,
          pltpu.SemaphoreType.DMA,
          pltpu.SemaphoreType.DMA,
      ],
  )
  return f(idx, val, bias16, tab)


def kernel(index, field, value, table, bias):
  del field
  idx = index.astype(jnp.int32).T
  val = value.T
  tab = jnp.pad(table, ((0, VOCAB_PAD - VOCAB), (0, 0)))[:, 0]
  bias16 = jnp.broadcast_to(bias.astype(jnp.float32), (16,))
  out = _wide_sc(idx, val, bias16, tab)
  return out.reshape(B, 1)


# speedup vs baseline: 1.1038x; 1.0290x over previous
import functools

import jax
import jax.numpy as jnp
from jax import lax
from jax.experimental import pallas as pl
from jax.experimental.pallas import tpu as pltpu
from jax.experimental.pallas import tpu_sc as plsc

B = 16384
F = 100
VOCAB = 1000000
VOCAB_PAD = 1000448

NC = 2
NS = 16
NW = NC * NS

W = 128
EPC = W * F
NCHUNK = B // W
CPW = NCHUNK // NW

TAB_SLICE = VOCAB_PAD // NS


def _body(idx_hbm, val_hbm, bias_hbm, table_hbm, out_hbm,
          idx_v0, idx_v1, val_v0, val_v1, gat_v, out_v, bias_v, tab_s,
          si0, si1, sv0, sv1, sg, sa, sb):
  sid = lax.axis_index("s")
  wid = sid * NC + lax.axis_index("c")
  pltpu.sync_copy(bias_hbm, bias_v)
  bias_vec = bias_v[...]
  idx_bufs = (idx_v0, idx_v1)
  val_bufs = (val_v0, val_v1)
  idx_sems = (si0, si1)
  val_sems = (sv0, sv1)

  def idx_start(k, s):
    c0 = (wid * CPW + k) * W
    ib = idx_bufs[s]

    def issue(f, carry):
      pltpu.make_async_copy(
          idx_hbm.at[f, pl.ds(c0, W)], ib.at[pl.ds(f * W, W)], idx_sems[s]
      ).start()
      return carry

    lax.fori_loop(0, F, issue, 0)

  def val_start(k, s):
    c0 = (wid * CPW + k) * W
    vb = val_bufs[s]

    def issue(f, carry):
      pltpu.make_async_copy(
          val_hbm.at[f, pl.ds(c0, W)], vb.at[pl.ds(f * W, W)], val_sems[s]
      ).start()
      return carry

    lax.fori_loop(0, F, issue, 0)

  def copies_start(k, s):
    idx_start(k, s)
    val_start(k, s)

  def idx_wait(k, s):
    pltpu.make_async_copy(
        idx_hbm.at[0, pl.ds(0, EPC)], idx_bufs[s], idx_sems[s]
    ).wait()

  def val_wait(k, s):
    pltpu.make_async_copy(
        val_hbm.at[0, pl.ds(0, EPC)], val_bufs[s], val_sems[s]
    ).wait()

  idx_start(0, 0)

  base = sid * TAB_SLICE
  pieces = [EPC] * 4 + [TAB_SLICE - 4 * EPC]
  offs = [sum(pieces[:i]) for i in range(len(pieces))]
  bounce = (gat_v, val_bufs[0], val_bufs[1])

  def _arr(i):
    return pltpu.make_async_copy(
        table_hbm.at[pl.ds(base + offs[i], pieces[i])],
        bounce[i % 3].at[pl.ds(0, pieces[i])], sa)

  def _wr(i):
    return pltpu.make_async_copy(
        bounce[i % 3].at[pl.ds(0, pieces[i])],
        tab_s.at[pl.ds(base + offs[i], pieces[i])], sb)

  n = len(pieces)
  for i in range(3):
    _arr(i).start()
  for i in range(n):
    _arr(i).wait()
    _wr(i).start()
    if i + 3 < n:
      _wr(i).wait()
      _arr(i + 3).start()
  for i in range(max(0, n - 3), n):
    _wr(i).wait()
  plsc.subcore_barrier()

  val_start(0, 0)
  HALF = EPC // 2

  def mk_ga(s):
    return pltpu.make_async_copy(
        tab_s.at[idx_bufs[s].at[pl.ds(0, HALF)]], gat_v.at[pl.ds(0, HALF)], sg)

  def mk_gb(s):
    return pltpu.make_async_copy(
        tab_s.at[idx_bufs[s].at[pl.ds(HALF, HALF)]],
        gat_v.at[pl.ds(HALF, HALF)], sg)

  idx_wait(0, 0)
  ga = mk_ga(0)
  ga.start()
  for k in range(CPW):
    s = k % 2
    vb = val_bufs[s]
    if k + 1 < CPW:
      copies_start(k + 1, 1 - s)
    val_wait(k, s)
    ga.wait()
    gb = mk_gb(s)
    gb.start()

    def fbody(f, accs):
      out = []
      for g in range(8):
        a = gat_v[pl.ds(f * W + g * 16, 16)]
        v = vb[pl.ds(f * W + g * 16, 16)]
        out.append(accs[g] + a * v)
      return tuple(out)

    accs = lax.fori_loop(
        0, F // 2, fbody, tuple(jnp.zeros((16,), jnp.float32) for _ in range(8))
    )
    gb.wait()
    if k + 1 < CPW:
      idx_wait(k + 1, 1 - s)
      ga = mk_ga(1 - s)
      ga.start()
    accs = lax.fori_loop(F // 2, F, fbody, accs)
    for g in range(8):
      out_v[pl.ds(g * 16, 16)] = accs[g] + bias_vec

    c0 = (wid * CPW + k) * W
    pltpu.sync_copy(out_v, out_hbm.at[pl.ds(c0, W)])


@jax.jit
def _wide_sc(idx, val, bias16, tab):
  mesh = plsc.VectorSubcoreMesh(core_axis_name="c", subcore_axis_name="s")
  f = pl.kernel(
      _body,
      mesh=mesh,
      compiler_params=pltpu.CompilerParams(needs_layout_passes=False),
      out_type=jax.ShapeDtypeStruct((B,), jnp.float32),
      scratch_types=[
          pltpu.VMEM((EPC,), jnp.int32),
          pltpu.VMEM((EPC,), jnp.int32),
          pltpu.VMEM((EPC,), jnp.float32),
          pltpu.VMEM((EPC,), jnp.float32),
          pltpu.VMEM((EPC,), jnp.float32),
          pltpu.VMEM((W,), jnp.float32),
          pltpu.VMEM((16,), jnp.float32),
          pltpu.VMEM_SHARED((VOCAB_PAD,), jnp.float32),
          pltpu.SemaphoreType.DMA,
          pltpu.SemaphoreType.DMA,
          pltpu.SemaphoreType.DMA,
          pltpu.SemaphoreType.DMA,
          pltpu.SemaphoreType.DMA,
          pltpu.SemaphoreType.DMA,
          pltpu.SemaphoreType.DMA,
      ],
  )
  return f(idx, val, bias16, tab)


def kernel(index, field, value, table, bias):
  del field
  idx = index.astype(jnp.int32).T
  val = value.T
  tab = jnp.pad(table, ((0, VOCAB_PAD - VOCAB), (0, 0)))[:, 0]
  bias16 = jnp.broadcast_to(bias.astype(jnp.float32), (16,))
  out = _wide_sc(idx, val, bias16, tab)
  return out.reshape(B, 1)
